# Initial kernel scaffold; baseline (speedup 1.0000x reference)
import jax
import jax.numpy as jnp
from jax.experimental import pallas as pl


def kernel(x, edge_index, batch, W_in, b_in, W1, b1, W2, b2, Wp, bp):
    raise NotImplementedError("write your pallas kernel here")


# speedup vs baseline: 1.1746x; 1.1746x over previous
import functools

import jax
import jax.numpy as jnp
from jax.experimental import pallas as pl
from jax.experimental.pallas import tpu as pltpu

_N = 50000
_G = 128
_BLK = 1000


def _linear_kernel(x_ref, w_ref, b_ref, o_ref, *, relu):
    y = jnp.dot(x_ref[...], w_ref[...], preferred_element_type=jnp.float32)
    y = y + b_ref[...]
    if relu:
        y = jnp.maximum(y, 0.0)
    o_ref[...] = y


def _linear(x, w, b, relu=False):
    n, k = x.shape
    m = w.shape[1]
    grid = n // _BLK
    return pl.pallas_call(
        functools.partial(_linear_kernel, relu=relu),
        grid=(grid,),
        in_specs=[
            pl.BlockSpec((_BLK, k), lambda i: (i, 0)),
            pl.BlockSpec((k, m), lambda i: (0, 0)),
            pl.BlockSpec((1, m), lambda i: (0, 0)),
        ],
        out_specs=pl.BlockSpec((_BLK, m), lambda i: (i, 0)),
        out_shape=jax.ShapeDtypeStruct((n, m), jnp.float32),
    )(x, w, b.reshape(1, m))


def _pool_kernel(h_ref, batch_ref, wp_ref, bp_ref, o_ref, hsum, hmax, cnt):
    i = pl.program_id(0)
    nb = pl.num_programs(0)

    @pl.when(i == 0)
    def _init():
        hsum[...] = jnp.zeros_like(hsum)
        hmax[...] = jnp.full_like(hmax, -1e30)
        cnt[...] = jnp.zeros_like(cnt)

    h = h_ref[...]
    b = batch_ref[...]
    gids = jax.lax.broadcasted_iota(jnp.int32, (1, _G), 1)
    onehot = (b == gids).astype(jnp.float32)
    hsum[...] += jnp.dot(onehot.T, h, preferred_element_type=jnp.float32)
    cnt[...] += jnp.sum(onehot, axis=0, keepdims=True)

    def body(g, _):
        mask = b[:, 0] == g
        m = jnp.max(jnp.where(mask[:, None], h, -1e30), axis=0)
        old = hmax[pl.ds(g, 1), :]
        hmax[pl.ds(g, 1), :] = jnp.maximum(old, m[None, :])
        return 0

    jax.lax.fori_loop(0, _G, body, 0)

    @pl.when(i == nb - 1)
    def _finish():
        c = cnt[...]
        mean = hsum[...] / jnp.maximum(c, 1.0).T
        mx = jnp.where(c.T > 0, hmax[...], 0.0)
        hg = jnp.concatenate([mean, mx], axis=1)
        y = jnp.dot(hg, wp_ref[...], preferred_element_type=jnp.float32)
        o_ref[...] = jnp.maximum(y + bp_ref[...], 0.0)


def _pool_project(h2, batch, Wp, bp):
    grid = _N // _BLK
    latent = Wp.shape[1]
    return pl.pallas_call(
        _pool_kernel,
        grid=(grid,),
        in_specs=[
            pl.BlockSpec((_BLK, h2.shape[1]), lambda i: (i, 0)),
            pl.BlockSpec((_BLK, 1), lambda i: (i, 0)),
            pl.BlockSpec(Wp.shape, lambda i: (0, 0)),
            pl.BlockSpec((1, latent), lambda i: (0, 0)),
        ],
        out_specs=pl.BlockSpec((_G, latent), lambda i: (0, 0)),
        out_shape=jax.ShapeDtypeStruct((_G, latent), jnp.float32),
        scratch_shapes=[
            pltpu.VMEM((_G, h2.shape[1]), jnp.float32),
            pltpu.VMEM((_G, h2.shape[1]), jnp.float32),
            pltpu.VMEM((1, _G), jnp.float32),
        ],
    )(h2, batch.reshape(_N, 1), Wp, bp.reshape(1, latent))


def _gcn(h, W, b, src, dst, dis):
    hW = _linear(h, W, jnp.zeros((W.shape[1],), jnp.float32))
    norm = dis[src] * dis[dst]
    msg = hW[src] * norm[:, None]
    out = jax.ops.segment_sum(msg, dst, num_segments=_N)
    out = out + hW * (dis * dis)[:, None]
    return out + b


def kernel(x, edge_index, batch, W_in, b_in, W1, b1, W2, b2, Wp, bp):
    src = edge_index[0]
    dst = edge_index[1]
    ones = jnp.ones((src.shape[0],), jnp.float32)
    deg = jax.ops.segment_sum(ones, dst, num_segments=_N) + 1.0
    dis = jnp.where(deg > 0, 1.0 / jnp.sqrt(jnp.maximum(deg, 1e-12)), 0.0)

    nf = _linear(x, W_in, b_in)
    h1 = jnp.maximum(_gcn(nf, W1, b1, src, dst, dis), 0.0)
    h2 = jnp.maximum(_gcn(h1, W2, b2, src, dst, dis), 0.0)
    return _pool_project(h2, batch, Wp, bp)
